# Initial kernel scaffold; baseline (speedup 1.0000x reference)
import jax
import jax.numpy as jnp
from jax import lax
from jax.experimental import pallas as pl
from jax.experimental.pallas import tpu as pltpu
from jax.experimental.pallas import tpu_sc as plsc

N = 10000
E = 320000
D = 128
H = 8
HD = 16

N1 = 10240
ROWS_PER_TILE = N1 // 16
C = 128
NCHUNK = 79
EPT = NCHUNK * C
E_PAD = EPT * 32
W = D + 2 * H


def _elu(v):
    return jnp.where(v > 0, v, jnp.exp(jnp.minimum(v, 0.0)) - 1.0)


def _prep_body(x_ref, wct_ref, bc_ref, al_ref, ar_ref, ab_ref,
               src_tab_ref, el_tab_ref):
    h = jnp.dot(x_ref[...], wct_ref[...], preferred_element_type=jnp.float32)
    h = h + bc_ref[...]
    el = jnp.dot(h, al_ref[...], preferred_element_type=jnp.float32) + ab_ref[...]
    er = jnp.dot(h, ar_ref[...], preferred_element_type=jnp.float32)
    r = h.shape[0]
    z8 = jnp.zeros((r, 8), jnp.float32)
    src_tab_ref[...] = jnp.concatenate([h, er, z8], axis=1)
    el_tab_ref[...] = jnp.concatenate([el, z8], axis=1)


def _prep(x_pad, wct, bc, al, ar, ab_row):
    blk = 512
    grid = (N1 // blk,)
    return pl.pallas_call(
        _prep_body,
        grid=grid,
        in_specs=[
            pl.BlockSpec((blk, D), lambda i: (i, 0)),
            pl.BlockSpec((D, D), lambda i: (0, 0)),
            pl.BlockSpec((1, D), lambda i: (0, 0)),
            pl.BlockSpec((D, H), lambda i: (0, 0)),
            pl.BlockSpec((D, H), lambda i: (0, 0)),
            pl.BlockSpec((1, H), lambda i: (0, 0)),
        ],
        out_specs=[
            pl.BlockSpec((blk, W), lambda i: (i, 0)),
            pl.BlockSpec((blk, 16), lambda i: (i, 0)),
        ],
        out_shape=[
            jax.ShapeDtypeStruct((N1, W), jnp.float32),
            jax.ShapeDtypeStruct((N1, 16), jnp.float32),
        ],
    )(x_pad, wct, bc, al, ar, ab_row)


def _edge_body(tgt_hbm, src_hbm, el_tab, src_tab, acc_out,
               idx_t, idx_s, el_rows, src_rows, out_rows, exmat, zbuf,
               sem_a, sem_b, acc_sh):
    c = lax.axis_index("c")
    s = lax.axis_index("s")

    zero16 = jnp.zeros((16,), jnp.float32)
    for r in range(16):
        for k in range(W // 16):
            zbuf[r, pl.ds(k * 16, 16)] = zero16

    def zloop(i, carry):
        pltpu.sync_copy(zbuf, acc_sh.at[pl.ds(s * ROWS_PER_TILE + i * 16, 16)])
        return carry
    lax.fori_loop(0, ROWS_PER_TILE // 16, zloop, 0)
    plsc.subcore_barrier()

    wid = s * 2 + c
    base = wid * EPT
    lanes = lax.iota(jnp.int32, 16)
    lane_mask = lanes < H
    lane_clamped = jnp.minimum(lanes, H - 1)

    def chunk(g, carry):
        off = base + g * C
        pltpu.sync_copy(tgt_hbm.at[pl.ds(off, C)], idx_t)
        pltpu.sync_copy(src_hbm.at[pl.ds(off, C)], idx_s)
        cp1 = pltpu.async_copy(el_tab.at[idx_t], el_rows, sem_a)
        cp2 = pltpu.async_copy(src_tab.at[idx_s], src_rows, sem_b)
        cp1.wait()
        cp2.wait()

        def logit_grp(j16, carry2):
            rowidx = j16 * 16 + lanes
            for hh in range(H):
                elh = plsc.load_gather(
                    el_rows, [rowidx, jnp.full((16,), hh, jnp.int32)])
                erh = plsc.load_gather(
                    src_rows, [rowidx, jnp.full((16,), D + hh, jnp.int32)])
                e = elh + erh
                e = jnp.where(e >= 0, e, 0.2 * e)
                exmat[hh, pl.ds(j16 * 16, 16)] = jnp.exp(e)
            return carry2
        lax.fori_loop(0, C // 16, logit_grp, 0)

        def edge(j, carry2):
            for hh in range(H):
                exs = exmat[hh, j]
                hj = src_rows[j, pl.ds(hh * 16, 16)]
                out_rows[j, pl.ds(hh * 16, 16)] = hj * exs
            tv = plsc.load_gather(
                exmat, [lane_clamped, jnp.full((16,), j, jnp.int32)])
            tv = jnp.where(lane_mask, tv, 0.0)
            out_rows[j, pl.ds(D, 16)] = tv
            return carry2
        lax.fori_loop(0, C, edge, 0)

        pltpu.sync_copy(out_rows, acc_sh.at[idx_t], add=True)
        return carry
    lax.fori_loop(0, NCHUNK, chunk, 0)
    plsc.subcore_barrier()

    pltpu.sync_copy(acc_sh.at[pl.ds(s * ROWS_PER_TILE, ROWS_PER_TILE)],
                    acc_out.at[c, pl.ds(s * ROWS_PER_TILE, ROWS_PER_TILE)])


def _edge_pass(tgt_pad, src_pad, el_tab, src_tab):
    mesh = plsc.VectorSubcoreMesh(core_axis_name="c", subcore_axis_name="s")
    f = pl.kernel(
        _edge_body,
        out_type=jax.ShapeDtypeStruct((2, N1, W), jnp.float32),
        mesh=mesh,
        scratch_types=[
            pltpu.VMEM((C,), jnp.int32),
            pltpu.VMEM((C,), jnp.int32),
            pltpu.VMEM((C, 16), jnp.float32),
            pltpu.VMEM((C, W), jnp.float32),
            pltpu.VMEM((C, W), jnp.float32),
            pltpu.VMEM((H, C), jnp.float32),
            pltpu.VMEM((16, W), jnp.float32),
            pltpu.SemaphoreType.DMA,
            pltpu.SemaphoreType.DMA,
            pltpu.VMEM_SHARED((N1, W), jnp.float32),
        ],
    )
    return f(tgt_pad, src_pad, el_tab, src_tab)


def _epi_body(acc0_ref, acc1_ref, src_tab_ref, b_ref, gm_ref, k_ref,
              gam_ref, bet_ref, ob_ref, out_ref):
    a = acc0_ref[0] + acc1_ref[0]
    num = a[:, :D]
    den = a[:, D:D + H]
    deninv = jnp.where(den > 0, 1.0 / den, 0.0)
    shb = jnp.dot(deninv, b_ref[...], preferred_element_type=jnp.float32)
    h = src_tab_ref[:, :D]
    y = _elu(num * shb + h)
    mu = jnp.dot(y, gm_ref[...], preferred_element_type=jnp.float32)
    d = y - mu
    var = jnp.dot(d * d, gm_ref[...], preferred_element_type=jnp.float32)
    z = d * lax.rsqrt(var + 1e-5) * gam_ref[...] + bet_ref[...]
    o = jnp.dot(z, k_ref[...], preferred_element_type=jnp.float32) + ob_ref[...]
    out_ref[...] = _elu(o)


def _epilogue(acc, src_tab, b_mat, gm, k_mat, gam, bet, ob_row):
    blk = 1000
    grid = (N // blk,)
    return pl.pallas_call(
        _epi_body,
        grid=grid,
        in_specs=[
            pl.BlockSpec((1, blk, W), lambda i: (0, i, 0)),
            pl.BlockSpec((1, blk, W), lambda i: (1, i, 0)),
            pl.BlockSpec((blk, W), lambda i: (i, 0)),
            pl.BlockSpec((H, D), lambda i: (0, 0)),
            pl.BlockSpec((D, D), lambda i: (0, 0)),
            pl.BlockSpec((D, D), lambda i: (0, 0)),
            pl.BlockSpec((1, D), lambda i: (0, 0)),
            pl.BlockSpec((1, D), lambda i: (0, 0)),
            pl.BlockSpec((1, D), lambda i: (0, 0)),
        ],
        out_specs=pl.BlockSpec((blk, D), lambda i: (i, 0)),
        out_shape=jax.ShapeDtypeStruct((N, D), jnp.float32),
    )(acc, acc, src_tab, b_mat, gm, k_mat, gam, bet, ob_row)


def kernel(node_features, edge_index, w_weight, w_bias, attn_weight, attn_bias,
           ln_gamma, ln_beta, out_weight, out_bias):
    f32 = jnp.float32
    wct = w_weight.reshape(H * HD, D).T.astype(f32)
    bc = w_bias.reshape(1, H * HD).astype(f32)
    a_l = attn_weight[:, 0, :HD]
    a_r = attn_weight[:, 0, HD:]
    eye8 = jnp.eye(H, dtype=f32)
    al = (a_l[:, :, None] * eye8[:, None, :]).reshape(H * HD, H)
    ar = (a_r[:, :, None] * eye8[:, None, :]).reshape(H * HD, H)
    ab_row = attn_bias[:, 0].reshape(1, H).astype(f32)
    b_mat = jnp.kron(eye8, jnp.ones((1, HD), f32))
    gm = jnp.kron(eye8, jnp.ones((HD, HD), f32) / HD)
    k_mat = jnp.tile(out_weight.T.astype(f32), (H, 1)) / H
    gam = ln_gamma.reshape(1, H * HD).astype(f32)
    bet = ln_beta.reshape(1, H * HD).astype(f32)
    ob_row = out_bias.reshape(1, D).astype(f32)

    x_pad = jnp.concatenate(
        [node_features.astype(f32), jnp.zeros((N1 - N, D), f32)], axis=0)
    pad_idx = jnp.full((E_PAD - E,), N, jnp.int32)
    tgt_pad = jnp.concatenate([edge_index[0].astype(jnp.int32), pad_idx])
    src_pad = jnp.concatenate([edge_index[1].astype(jnp.int32), pad_idx])

    src_tab, el_tab = _prep(x_pad, wct, bc, al, ar, ab_row)
    acc = _edge_pass(tgt_pad, src_pad, el_tab, src_tab)
    return _epilogue(acc, src_tab, b_mat, gm, k_mat, gam, bet, ob_row)


# speedup vs baseline: 137.7343x; 137.7343x over previous
import jax
import jax.numpy as jnp
from jax import lax
from jax.experimental import pallas as pl
from jax.experimental.pallas import tpu as pltpu
from jax.experimental.pallas import tpu_sc as plsc

N = 10000
E = 320000
D = 128
H = 8
HD = 16

N1 = 10240
ROWS_PER_TILE = N1 // 16
C = 80
NCHUNK = 126
EPT = NCHUNK * C
E_PAD = EPT * 32
W = D + 2 * H


def _elu(v):
    return jnp.where(v > 0, v, jnp.exp(jnp.minimum(v, 0.0)) - 1.0)


def _prep_body(x_ref, wct_ref, wctp_ref, bc_ref, bcp_ref, al_ref, ar_ref,
               ab_ref, pint_ref, src_tab_ref, el_tab_ref, hp_tab_ref):
    h = jnp.dot(x_ref[...], wct_ref[...], preferred_element_type=jnp.float32)
    h = h + bc_ref[...]
    hp = jnp.dot(x_ref[...], wctp_ref[...], preferred_element_type=jnp.float32)
    hp_tab_ref[...] = hp + bcp_ref[...]
    el = jnp.dot(h, al_ref[...], preferred_element_type=jnp.float32) + ab_ref[...]
    er = jnp.dot(h, ar_ref[...], preferred_element_type=jnp.float32)
    eri = jnp.dot(er, pint_ref[...], preferred_element_type=jnp.float32)
    r = h.shape[0]
    z8 = jnp.zeros((r, 8), jnp.float32)
    src_tab_ref[...] = jnp.concatenate([h, eri], axis=1).astype(jnp.bfloat16)
    el_tab_ref[...] = jnp.concatenate([el, z8], axis=1)


def _prep(x_pad, wct, wctp, bc, bcp, al, ar, ab_row, pint):
    blk = 512
    grid = (N1 // blk,)
    return pl.pallas_call(
        _prep_body,
        grid=grid,
        in_specs=[
            pl.BlockSpec((blk, D), lambda i: (i, 0)),
            pl.BlockSpec((D, D), lambda i: (0, 0)),
            pl.BlockSpec((D, D), lambda i: (0, 0)),
            pl.BlockSpec((1, D), lambda i: (0, 0)),
            pl.BlockSpec((1, D), lambda i: (0, 0)),
            pl.BlockSpec((D, H), lambda i: (0, 0)),
            pl.BlockSpec((D, H), lambda i: (0, 0)),
            pl.BlockSpec((1, H), lambda i: (0, 0)),
            pl.BlockSpec((H, 32), lambda i: (0, 0)),
        ],
        out_specs=[
            pl.BlockSpec((blk, 160), lambda i: (i, 0)),
            pl.BlockSpec((blk, 16), lambda i: (i, 0)),
            pl.BlockSpec((blk, D), lambda i: (i, 0)),
        ],
        out_shape=[
            jax.ShapeDtypeStruct((N1, 160), jnp.bfloat16),
            jax.ShapeDtypeStruct((N1, 16), jnp.float32),
            jax.ShapeDtypeStruct((N1, D), jnp.float32),
        ],
    )(x_pad, wct, wctp, bc, bcp, al, ar, ab_row, pint)


def _edge_body(edges3, el_tab, src_tab, acc_out,
               idx0, idx1, el0, el1, sb0, sb1, ob0, ob1, oidx0, oidx1,
               sem_e0, sem_e1, sem_s0, sem_s1, sem_c0, sem_c1, acc_sh):
    c = lax.axis_index("c")
    s = lax.axis_index("s")
    lanes = lax.iota(jnp.int32, 16)
    lane_mask = lanes < H
    ITL = plsc.PackFormat.INTERLEAVED

    zero16 = jnp.zeros((16,), jnp.float32)

    def zrow(j, carry):
        for k in range(W // 16):
            ob1[j, pl.ds(k * 16, 16)] = zero16
        return carry
    lax.fori_loop(0, C, zrow, 0)
    zbase = s * ROWS_PER_TILE
    for t in range(ROWS_PER_TILE // C):
        pltpu.sync_copy(ob1, acc_sh.at[pl.ds(zbase + t * C, C)])
    rem = ROWS_PER_TILE % C
    if rem:
        pltpu.sync_copy(ob1.at[pl.ds(0, rem)],
                        acc_sh.at[pl.ds(zbase + (ROWS_PER_TILE // C) * C, rem)])

    wid = s * 2 + c
    pbase = wid * NCHUNK

    def gathers(k, idxb, elb, sbb, sem_e, sem_s):
        pltpu.sync_copy(edges3.at[:, k], idxb)
        pltpu.async_copy(el_tab.at[idxb.at[0]], elb, sem_e)
        pltpu.async_copy(src_tab.at[idxb.at[1]], sbb, sem_s)

    def wait_gathers(idxb, elb, sbb, sem_e, sem_s):
        pltpu.make_async_copy(el_tab.at[idxb.at[0]], elb, sem_e).wait()
        pltpu.make_async_copy(src_tab.at[idxb.at[1]], sbb, sem_s).wait()

    def copy_oidx(idxb, oidxb):
        for k in range(C // 16):
            oidxb[pl.ds(k * 16, 16)] = idxb[0, pl.ds(k * 16, 16)]

    def scatter(obb, oidxb, sem_c):
        pltpu.async_copy(obb, acc_sh.at[oidxb], sem_c, add=True)

    def wait_scatter(obb, oidxb, sem_c):
        pltpu.make_async_copy(obb, acc_sh.at[oidxb], sem_c).wait()

    def compute(elb, sbb, obb):
        @plsc.parallel_loop(0, C, unroll=2)
        def edge(j):
            ervec, _ = plsc.unpack(sbb[j, pl.ds(D, 32)], format=ITL)
            e = elb[j, pl.ds(0, 16)] + ervec
            e = jnp.where(e >= 0, e, 0.2 * e)
            tv = jnp.where(lane_mask, jnp.exp(e), 0.0)
            for g in range(4):
                a, b = plsc.unpack(sbb[j, pl.ds(32 * g, 32)], format=ITL)
                m = jnp.where(lane_mask, tv[2 * g], tv[2 * g + 1])
                obb[j, pl.ds(32 * g, 16)] = a * m
                obb[j, pl.ds(32 * g + 16, 16)] = b * m
            obb[j, pl.ds(D, 16)] = tv

    gathers(pbase, idx0, el0, sb0, sem_e0, sem_s0)
    plsc.subcore_barrier()

    def pair(p, carry):
        k0 = pbase + 2 * p

        gathers(k0 + 1, idx1, el1, sb1, sem_e1, sem_s1)

        wait_gathers(idx0, el0, sb0, sem_e0, sem_s0)

        @pl.when(p > 0)
        def _():
            wait_scatter(ob0, oidx0, sem_c0)
        copy_oidx(idx0, oidx0)
        compute(el0, sb0, ob0)
        scatter(ob0, oidx0, sem_c0)

        @pl.when(p < NCHUNK // 2 - 1)
        def _():
            gathers(k0 + 2, idx0, el0, sb0, sem_e0, sem_s0)

        wait_gathers(idx1, el1, sb1, sem_e1, sem_s1)

        @pl.when(p > 0)
        def _():
            wait_scatter(ob1, oidx1, sem_c1)
        copy_oidx(idx1, oidx1)
        compute(el1, sb1, ob1)
        scatter(ob1, oidx1, sem_c1)
        return carry
    lax.fori_loop(0, NCHUNK // 2, pair, 0)
    wait_scatter(ob0, oidx0, sem_c0)
    wait_scatter(ob1, oidx1, sem_c1)
    plsc.subcore_barrier()

    pltpu.sync_copy(acc_sh.at[pl.ds(s * ROWS_PER_TILE, ROWS_PER_TILE)],
                    acc_out.at[c, pl.ds(s * ROWS_PER_TILE, ROWS_PER_TILE)])


def _edge_pass(edges3, el_tab, src_tab):
    mesh = plsc.VectorSubcoreMesh(core_axis_name="c", subcore_axis_name="s")
    f = pl.kernel(
        _edge_body,
        out_type=jax.ShapeDtypeStruct((2, N1, W), jnp.float32),
        mesh=mesh,
        compiler_params=pltpu.CompilerParams(
            use_tc_tiling_on_sc=False, needs_layout_passes=False),
        scratch_types=(
            [pltpu.VMEM((2, C), jnp.int32)] * 2
            + [pltpu.VMEM((C, 16), jnp.float32)] * 2
            + [pltpu.VMEM((C, 160), jnp.bfloat16)] * 2
            + [pltpu.VMEM((C, W), jnp.float32)] * 2
            + [pltpu.VMEM((C,), jnp.int32)] * 2
            + [pltpu.SemaphoreType.DMA] * 6
            + [pltpu.VMEM_SHARED((N1, W), jnp.float32)]
        ),
    )
    return f(edges3, el_tab, src_tab)


def _epi_body(acc0_ref, acc1_ref, hp_ref, b_ref, gm_ref, k_ref,
              gam_ref, bet_ref, ob_ref, out_ref):
    a = acc0_ref[0] + acc1_ref[0]
    num = a[:, :D]
    den = a[:, D:D + H]
    deninv = jnp.where(den > 0, 1.0 / den, 0.0)
    shb = jnp.dot(deninv, b_ref[...], preferred_element_type=jnp.float32)
    h = hp_ref[...]
    y = _elu(num * shb + h)
    mu = jnp.dot(y, gm_ref[...], preferred_element_type=jnp.float32)
    d = y - mu
    var = jnp.dot(d * d, gm_ref[...], preferred_element_type=jnp.float32)
    z = d * lax.rsqrt(var + 1e-5) * gam_ref[...] + bet_ref[...]
    o = jnp.dot(z, k_ref[...], preferred_element_type=jnp.float32) + ob_ref[...]
    out_ref[...] = _elu(o)


def _epilogue(acc, hp_tab, b_mat, gm, k_mat, gam, bet, ob_row):
    blk = 1000
    grid = (N // blk,)
    return pl.pallas_call(
        _epi_body,
        grid=grid,
        in_specs=[
            pl.BlockSpec((1, blk, W), lambda i: (0, i, 0)),
            pl.BlockSpec((1, blk, W), lambda i: (1, i, 0)),
            pl.BlockSpec((blk, D), lambda i: (i, 0)),
            pl.BlockSpec((H, D), lambda i: (0, 0)),
            pl.BlockSpec((D, D), lambda i: (0, 0)),
            pl.BlockSpec((D, D), lambda i: (0, 0)),
            pl.BlockSpec((1, D), lambda i: (0, 0)),
            pl.BlockSpec((1, D), lambda i: (0, 0)),
            pl.BlockSpec((1, D), lambda i: (0, 0)),
        ],
        out_specs=pl.BlockSpec((blk, D), lambda i: (i, 0)),
        out_shape=jax.ShapeDtypeStruct((N, D), jnp.float32),
    )(acc, acc, hp_tab, b_mat, gm, k_mat, gam, bet, ob_row)


def kernel(node_features, edge_index, w_weight, w_bias, attn_weight, attn_bias,
           ln_gamma, ln_beta, out_weight, out_bias):
    f32 = jnp.float32
    wct = w_weight.reshape(H * HD, D).T.astype(f32)
    bc = w_bias.reshape(1, H * HD).astype(f32)
    a_l = attn_weight[:, 0, :HD]
    a_r = attn_weight[:, 0, HD:]
    eye8 = jnp.eye(H, dtype=f32)
    al = (a_l[:, :, None] * eye8[:, None, :]).reshape(H * HD, H)
    ar = (a_r[:, :, None] * eye8[:, None, :]).reshape(H * HD, H)
    ab_row = attn_bias[:, 0].reshape(1, H).astype(f32)
    b_mat = jnp.kron(eye8, jnp.ones((1, HD), f32))
    gm = jnp.kron(eye8, jnp.ones((HD, HD), f32) / HD)
    k_mat = jnp.tile(out_weight.T.astype(f32), (H, 1)) / H
    gam = ln_gamma.reshape(1, H * HD).astype(f32)
    bet = ln_beta.reshape(1, H * HD).astype(f32)
    ob_row = out_bias.reshape(1, D).astype(f32)

    q = []
    for g in range(4):
        q += [32 * g + 2 * i for i in range(16)]
        q += [32 * g + 2 * i + 1 for i in range(16)]
    qa = jnp.array(q, jnp.int32)
    wctp = wct[:, qa]
    bcp = bc[:, qa]
    b_perm = b_mat[:, qa]
    gm_perm = gm[qa][:, qa]
    k_perm = k_mat[qa, :]
    gam_p = gam[:, qa]
    bet_p = bet[:, qa]
    pint = jnp.zeros((H, 32), f32).at[jnp.arange(H), 2 * jnp.arange(H)].set(1.0)

    x_pad = jnp.concatenate(
        [node_features.astype(f32), jnp.zeros((N1 - N, D), f32)], axis=0)
    pad_idx = jnp.full((2, E_PAD - E), N, jnp.int32)
    edges3 = jnp.concatenate(
        [edge_index.astype(jnp.int32), pad_idx], axis=1).reshape(2, -1, C)

    src_tab, el_tab, hp_tab = _prep(x_pad, wct, wctp, bc, bcp, al, ar,
                                    ab_row, pint)
    acc = _edge_pass(edges3, el_tab, src_tab)
    return _epilogue(acc, hp_tab, b_perm, gm_perm, k_perm, gam_p, bet_p,
                     ob_row)


# speedup vs baseline: 137.9377x; 1.0015x over previous
import jax
import jax.numpy as jnp
from jax import lax
from jax.experimental import pallas as pl
from jax.experimental.pallas import tpu as pltpu
from jax.experimental.pallas import tpu_sc as plsc

N = 10000
E = 320000
D = 128
H = 8
HD = 16

N1 = 10240
ROWS_PER_TILE = N1 // 16
C = 80
NCHUNK = 126
EPT = NCHUNK * C
E_PAD = EPT * 32
W = D + 2 * H


def _elu(v):
    return jnp.where(v > 0, v, jnp.exp(jnp.minimum(v, 0.0)) - 1.0)


def _prep_body(x_ref, wct_ref, wctp_ref, bc_ref, bcp_ref, al_ref, ar_ref,
               ab_ref, pint_ref, src_tab_ref, el_tab_ref, hp_tab_ref):
    h = jnp.dot(x_ref[...], wct_ref[...], preferred_element_type=jnp.float32)
    h = h + bc_ref[...]
    hp = jnp.dot(x_ref[...], wctp_ref[...], preferred_element_type=jnp.float32)
    hp_tab_ref[...] = hp + bcp_ref[...]
    el = jnp.dot(h, al_ref[...], preferred_element_type=jnp.float32) + ab_ref[...]
    er = jnp.dot(h, ar_ref[...], preferred_element_type=jnp.float32)
    eri = jnp.dot(er, pint_ref[...], preferred_element_type=jnp.float32)
    r = h.shape[0]
    z8 = jnp.zeros((r, 8), jnp.float32)
    src_tab_ref[...] = jnp.concatenate([h, eri], axis=1).astype(jnp.bfloat16)
    el_tab_ref[...] = jnp.concatenate([el, z8], axis=1)


def _prep(x_pad, wct, wctp, bc, bcp, al, ar, ab_row, pint):
    blk = 512
    grid = (N1 // blk,)
    return pl.pallas_call(
        _prep_body,
        grid=grid,
        in_specs=[
            pl.BlockSpec((blk, D), lambda i: (i, 0)),
            pl.BlockSpec((D, D), lambda i: (0, 0)),
            pl.BlockSpec((D, D), lambda i: (0, 0)),
            pl.BlockSpec((1, D), lambda i: (0, 0)),
            pl.BlockSpec((1, D), lambda i: (0, 0)),
            pl.BlockSpec((D, H), lambda i: (0, 0)),
            pl.BlockSpec((D, H), lambda i: (0, 0)),
            pl.BlockSpec((1, H), lambda i: (0, 0)),
            pl.BlockSpec((H, 32), lambda i: (0, 0)),
        ],
        out_specs=[
            pl.BlockSpec((blk, 160), lambda i: (i, 0)),
            pl.BlockSpec((blk, 16), lambda i: (i, 0)),
            pl.BlockSpec((blk, D), lambda i: (i, 0)),
        ],
        out_shape=[
            jax.ShapeDtypeStruct((N1, 160), jnp.bfloat16),
            jax.ShapeDtypeStruct((N1, 16), jnp.float32),
            jax.ShapeDtypeStruct((N1, D), jnp.float32),
        ],
    )(x_pad, wct, wctp, bc, bcp, al, ar, ab_row, pint)


def _edge_body(edges3, el_tab, src_tab, acc_out,
               idx0, idx1, el0, el1, sb0, sb1, ob0, ob1, oidx0, oidx1,
               sem_e0, sem_e1, sem_s0, sem_s1, sem_c0, sem_c1, acc_sh):
    c = lax.axis_index("c")
    s = lax.axis_index("s")
    lanes = lax.iota(jnp.int32, 16)
    lane_mask = lanes < H
    ITL = plsc.PackFormat.INTERLEAVED

    zero16 = jnp.zeros((16,), jnp.float32)

    def zrow(j, carry):
        for k in range(W // 16):
            ob1[j, pl.ds(k * 16, 16)] = zero16
        return carry
    lax.fori_loop(0, C, zrow, 0)
    zbase = s * ROWS_PER_TILE
    for t in range(ROWS_PER_TILE // C):
        pltpu.sync_copy(ob1, acc_sh.at[pl.ds(zbase + t * C, C)])
    rem = ROWS_PER_TILE % C
    if rem:
        pltpu.sync_copy(ob1.at[pl.ds(0, rem)],
                        acc_sh.at[pl.ds(zbase + (ROWS_PER_TILE // C) * C, rem)])

    wid = s * 2 + c
    pbase = wid * NCHUNK

    def gathers(k, idxb, elb, sbb, sem_e, sem_s):
        pltpu.sync_copy(edges3.at[:, k], idxb)
        pltpu.async_copy(el_tab.at[idxb.at[0]], elb, sem_e)
        pltpu.async_copy(src_tab.at[idxb.at[1]], sbb, sem_s)

    def wait_gathers(idxb, elb, sbb, sem_e, sem_s):
        pltpu.make_async_copy(el_tab.at[idxb.at[0]], elb, sem_e).wait()
        pltpu.make_async_copy(src_tab.at[idxb.at[1]], sbb, sem_s).wait()

    def copy_oidx(idxb, oidxb):
        for k in range(C // 16):
            oidxb[pl.ds(k * 16, 16)] = idxb[0, pl.ds(k * 16, 16)]

    def scatter(obb, oidxb, sem_c):
        pltpu.async_copy(obb, acc_sh.at[oidxb], sem_c, add=True)

    def wait_scatter(obb, oidxb, sem_c):
        pltpu.make_async_copy(obb, acc_sh.at[oidxb], sem_c).wait()

    def compute(elb, sbb, obb):
        @plsc.parallel_loop(0, C, unroll=4)
        def edge(j):
            ervec, _ = plsc.unpack(sbb[j, pl.ds(D, 32)], format=ITL)
            e = elb[j, pl.ds(0, 16)] + ervec
            e = jnp.where(e >= 0, e, 0.2 * e)
            tv = jnp.where(lane_mask, jnp.exp(e), 0.0)
            for g in range(4):
                a, b = plsc.unpack(sbb[j, pl.ds(32 * g, 32)], format=ITL)
                m = jnp.where(lane_mask, tv[2 * g], tv[2 * g + 1])
                obb[j, pl.ds(32 * g, 16)] = a * m
                obb[j, pl.ds(32 * g + 16, 16)] = b * m
            obb[j, pl.ds(D, 16)] = tv

    gathers(pbase, idx0, el0, sb0, sem_e0, sem_s0)
    plsc.subcore_barrier()

    def pair(p, carry):
        k0 = pbase + 2 * p

        gathers(k0 + 1, idx1, el1, sb1, sem_e1, sem_s1)

        wait_gathers(idx0, el0, sb0, sem_e0, sem_s0)

        @pl.when(p > 0)
        def _():
            wait_scatter(ob0, oidx0, sem_c0)
        copy_oidx(idx0, oidx0)
        compute(el0, sb0, ob0)
        scatter(ob0, oidx0, sem_c0)

        @pl.when(p < NCHUNK // 2 - 1)
        def _():
            gathers(k0 + 2, idx0, el0, sb0, sem_e0, sem_s0)

        wait_gathers(idx1, el1, sb1, sem_e1, sem_s1)

        @pl.when(p > 0)
        def _():
            wait_scatter(ob1, oidx1, sem_c1)
        copy_oidx(idx1, oidx1)
        compute(el1, sb1, ob1)
        scatter(ob1, oidx1, sem_c1)
        return carry
    lax.fori_loop(0, NCHUNK // 2, pair, 0)
    wait_scatter(ob0, oidx0, sem_c0)
    wait_scatter(ob1, oidx1, sem_c1)
    plsc.subcore_barrier()

    pltpu.sync_copy(acc_sh.at[pl.ds(s * ROWS_PER_TILE, ROWS_PER_TILE)],
                    acc_out.at[c, pl.ds(s * ROWS_PER_TILE, ROWS_PER_TILE)])


def _edge_pass(edges3, el_tab, src_tab):
    mesh = plsc.VectorSubcoreMesh(core_axis_name="c", subcore_axis_name="s")
    f = pl.kernel(
        _edge_body,
        out_type=jax.ShapeDtypeStruct((2, N1, W), jnp.float32),
        mesh=mesh,
        compiler_params=pltpu.CompilerParams(
            use_tc_tiling_on_sc=False, needs_layout_passes=False),
        scratch_types=(
            [pltpu.VMEM((2, C), jnp.int32)] * 2
            + [pltpu.VMEM((C, 16), jnp.float32)] * 2
            + [pltpu.VMEM((C, 160), jnp.bfloat16)] * 2
            + [pltpu.VMEM((C, W), jnp.float32)] * 2
            + [pltpu.VMEM((C,), jnp.int32)] * 2
            + [pltpu.SemaphoreType.DMA] * 6
            + [pltpu.VMEM_SHARED((N1, W), jnp.float32)]
        ),
    )
    return f(edges3, el_tab, src_tab)


def _epi_body(acc0_ref, acc1_ref, hp_ref, b_ref, gm_ref, k_ref,
              gam_ref, bet_ref, ob_ref, out_ref):
    a = acc0_ref[0] + acc1_ref[0]
    num = a[:, :D]
    den = a[:, D:D + H]
    deninv = jnp.where(den > 0, 1.0 / den, 0.0)
    shb = jnp.dot(deninv, b_ref[...], preferred_element_type=jnp.float32)
    h = hp_ref[...]
    y = _elu(num * shb + h)
    mu = jnp.dot(y, gm_ref[...], preferred_element_type=jnp.float32)
    d = y - mu
    var = jnp.dot(d * d, gm_ref[...], preferred_element_type=jnp.float32)
    z = d * lax.rsqrt(var + 1e-5) * gam_ref[...] + bet_ref[...]
    o = jnp.dot(z, k_ref[...], preferred_element_type=jnp.float32) + ob_ref[...]
    out_ref[...] = _elu(o)


def _epilogue(acc, hp_tab, b_mat, gm, k_mat, gam, bet, ob_row):
    blk = 1000
    grid = (N // blk,)
    return pl.pallas_call(
        _epi_body,
        grid=grid,
        in_specs=[
            pl.BlockSpec((1, blk, W), lambda i: (0, i, 0)),
            pl.BlockSpec((1, blk, W), lambda i: (1, i, 0)),
            pl.BlockSpec((blk, D), lambda i: (i, 0)),
            pl.BlockSpec((H, D), lambda i: (0, 0)),
            pl.BlockSpec((D, D), lambda i: (0, 0)),
            pl.BlockSpec((D, D), lambda i: (0, 0)),
            pl.BlockSpec((1, D), lambda i: (0, 0)),
            pl.BlockSpec((1, D), lambda i: (0, 0)),
            pl.BlockSpec((1, D), lambda i: (0, 0)),
        ],
        out_specs=pl.BlockSpec((blk, D), lambda i: (i, 0)),
        out_shape=jax.ShapeDtypeStruct((N, D), jnp.float32),
    )(acc, acc, hp_tab, b_mat, gm, k_mat, gam, bet, ob_row)


def kernel(node_features, edge_index, w_weight, w_bias, attn_weight, attn_bias,
           ln_gamma, ln_beta, out_weight, out_bias):
    f32 = jnp.float32
    wct = w_weight.reshape(H * HD, D).T.astype(f32)
    bc = w_bias.reshape(1, H * HD).astype(f32)
    a_l = attn_weight[:, 0, :HD]
    a_r = attn_weight[:, 0, HD:]
    eye8 = jnp.eye(H, dtype=f32)
    al = (a_l[:, :, None] * eye8[:, None, :]).reshape(H * HD, H)
    ar = (a_r[:, :, None] * eye8[:, None, :]).reshape(H * HD, H)
    ab_row = attn_bias[:, 0].reshape(1, H).astype(f32)
    b_mat = jnp.kron(eye8, jnp.ones((1, HD), f32))
    gm = jnp.kron(eye8, jnp.ones((HD, HD), f32) / HD)
    k_mat = jnp.tile(out_weight.T.astype(f32), (H, 1)) / H
    gam = ln_gamma.reshape(1, H * HD).astype(f32)
    bet = ln_beta.reshape(1, H * HD).astype(f32)
    ob_row = out_bias.reshape(1, D).astype(f32)

    q = []
    for g in range(4):
        q += [32 * g + 2 * i for i in range(16)]
        q += [32 * g + 2 * i + 1 for i in range(16)]
    qa = jnp.array(q, jnp.int32)
    wctp = wct[:, qa]
    bcp = bc[:, qa]
    b_perm = b_mat[:, qa]
    gm_perm = gm[qa][:, qa]
    k_perm = k_mat[qa, :]
    gam_p = gam[:, qa]
    bet_p = bet[:, qa]
    pint = jnp.zeros((H, 32), f32).at[jnp.arange(H), 2 * jnp.arange(H)].set(1.0)

    x_pad = jnp.concatenate(
        [node_features.astype(f32), jnp.zeros((N1 - N, D), f32)], axis=0)
    pad_idx = jnp.full((2, E_PAD - E), N, jnp.int32)
    edges3 = jnp.concatenate(
        [edge_index.astype(jnp.int32), pad_idx], axis=1).reshape(2, -1, C)

    src_tab, el_tab, hp_tab = _prep(x_pad, wct, wctp, bc, bcp, al, ar,
                                    ab_row, pint)
    acc = _edge_pass(edges3, el_tab, src_tab)
    return _epilogue(acc, hp_tab, b_perm, gm_perm, k_perm, gam_p, bet_p,
                     ob_row)
